# Initial kernel scaffold; baseline (speedup 1.0000x reference)
import jax
import jax.numpy as jnp
from jax.experimental import pallas as pl


def kernel(pos, tables):
    raise NotImplementedError("write your pallas kernel here")


# speedup vs baseline: 37.1915x; 37.1915x over previous
import functools
import math

import jax
import jax.numpy as jnp
import numpy as np
from jax import lax
from jax.experimental import pallas as pl
from jax.experimental.pallas import tpu as pltpu
from jax.experimental.pallas import tpu_sc as plsc

_R = [150, 600, 2400, 10000, 40000]
_NB = 64
_MAXD = 256
_ODIM = 16
_T = 2048
_L = 16
_NW = 32
_ROWS_PER_W = _T // _NW
_DPAD = _T + _L

_t = [0] * 32
for _k in range(1, 16):
    _t[_k] = _k
for _k in range(16, 32):
    _t[_k] = math.ceil(2 ** (_k / 4.0))

_s = np.zeros(64, dtype=np.int64)
_s[0] = -(1 << 30)
for _j in range(1, 32):
    _s[_j] = 1 - _t[32 - _j]
_s[32] = 0
for _j in range(33, 64):
    _s[_j] = _t[_j - 32]
_S_ARR = _s.astype(np.int32)
_B_OF_J = np.array([63 - j if j < 32 else j - 32 for j in range(64)])


def _lane_bcast(vec, j):
    idx = jnp.full((_L, 1), j, dtype=jnp.int32)
    dnums = lax.GatherDimensionNumbers(
        offset_dims=(), collapsed_slice_dims=(0,), start_index_map=(0,))
    return lax.gather(vec, idx, dnums, (1,),
                      mode=lax.GatherScatterMode.PROMISE_IN_BOUNDS)


def _body(pos_hbm, dt_hbm, s_hbm, out_hbm, pos_v, p_v, dt_v, s_v, delta_v, slab_v):
    nc = 2
    wid = lax.axis_index("s") * nc + lax.axis_index("c")
    row0 = wid * _ROWS_PER_W

    pltpu.sync_copy(pos_hbm, pos_v)
    pltpu.sync_copy(dt_hbm, dt_v)
    pltpu.sync_copy(s_hbm, s_v)

    def compute_p(c, _):
        v = pos_v[pl.ds(c * _L, _L)]
        for i in range(5):
            p_v[i, pl.ds(c * _L, _L)] = lax.div(v, jnp.int32(_R[i]))
        return 0

    lax.fori_loop(0, _T // _L, compute_p, 0)

    def clear(k, _):
        delta_v[k] = jnp.zeros((_L,), jnp.float32)
        return 0

    lax.fori_loop(0, _DPAD, clear, 0)

    o_iota = lax.iota(jnp.int32, _L)
    steps = [1024, 512, 256, 128, 64, 32, 16, 8, 4, 2, 1, 1]

    def do_row(k, _):
        row = row0 + k
        rowfull = jnp.full((_L,), 0, dtype=jnp.int32) + row
        for i in range(5):
            ifull = jnp.full((_L,), i, dtype=jnp.int32)
            prv = plsc.load_gather(p_v, [ifull, rowfull])
            for g in range(4):
                tgt = prv + s_v[pl.ds(g * _L, _L)]
                lo = jnp.zeros((_L,), jnp.int32)
                for st in steps:
                    cand = lo + st
                    pv = plsc.load_gather(p_v, [ifull, cand - 1])
                    lo = jnp.where(pv < tgt, cand, lo)
                for j in range(_L):
                    b = _lane_bcast(lo, j)
                    val = dt_v[i * 64 + g * _L + j]
                    plsc.addupdate_scatter(delta_v, [b, o_iota], val)
        zero16 = jnp.zeros((_L,), jnp.float32)

        def walk(col, acc):
            acc = acc + delta_v[col]
            delta_v[col] = zero16
            plsc.store_scatter(slab_v, [o_iota, jnp.full((_L,), 0, jnp.int32) + col], acc)
            return acc

        lax.fori_loop(0, _T, walk, zero16)
        pltpu.sync_copy(slab_v, out_hbm.at[:, row])
        return 0

    lax.fori_loop(0, _ROWS_PER_W, do_row, 0)


@jax.jit
def kernel(pos, tables):
    u = jnp.take(tables, jnp.asarray(_B_OF_J), axis=1)
    d = jnp.concatenate([u[:, :1], u[:, 1:] - u[:, :-1]], axis=1)
    dt = d.reshape(5 * 64, _ODIM).astype(jnp.float32)
    s_arr = jnp.asarray(_S_ARR)

    mesh = plsc.VectorSubcoreMesh(core_axis_name="c", subcore_axis_name="s")
    f = functools.partial(
        pl.kernel,
        mesh=mesh,
        compiler_params=pltpu.CompilerParams(
            needs_layout_passes=False, use_tc_tiling_on_sc=False),
        out_type=jax.ShapeDtypeStruct((_ODIM, _T, _T), jnp.float32),
        scratch_types=[
            pltpu.VMEM((_T,), jnp.int32),
            pltpu.VMEM((5, _T), jnp.int32),
            pltpu.VMEM((5 * 64, _ODIM), jnp.float32),
            pltpu.VMEM((64,), jnp.int32),
            pltpu.VMEM((_DPAD, _ODIM), jnp.float32),
            pltpu.VMEM((_ODIM, _T), jnp.float32),
        ],
    )(_body)
    return f(pos.astype(jnp.int32), dt, s_arr)


# speedup vs baseline: 39.4768x; 1.0614x over previous
import functools
import math

import jax
import jax.numpy as jnp
import numpy as np
from jax import lax
from jax.experimental import pallas as pl
from jax.experimental.pallas import tpu as pltpu
from jax.experimental.pallas import tpu_sc as plsc

_R = [150, 600, 2400, 10000, 40000]
_NB = 64
_MAXD = 256
_ODIM = 16
_T = 2048
_L = 16
_NW = 32
_ROWS_PER_W = _T // _NW
_DPAD = _T + _L

_t = [0] * 32
for _k in range(1, 16):
    _t[_k] = _k
for _k in range(16, 32):
    _t[_k] = math.ceil(2 ** (_k / 4.0))

_s = np.zeros(64, dtype=np.int64)
_s[0] = -(1 << 30)
for _j in range(1, 32):
    _s[_j] = 1 - _t[32 - _j]
_s[32] = 0
for _j in range(33, 64):
    _s[_j] = _t[_j - 32]
_S_ARR = _s.astype(np.int32)
_B_OF_J = np.array([63 - j if j < 32 else j - 32 for j in range(64)])


def _lane_bcast(vec, j):
    idx = jnp.full((_L, 1), j, dtype=jnp.int32)
    dnums = lax.GatherDimensionNumbers(
        offset_dims=(), collapsed_slice_dims=(0,), start_index_map=(0,))
    return lax.gather(vec, idx, dnums, (1,),
                      mode=lax.GatherScatterMode.PROMISE_IN_BOUNDS)


def _body(pos_hbm, dt_hbm, s_hbm, out_hbm, pos_v, p_v, dt_v, s_v, delta_v,
          slab0_v, slab1_v, sem0, sem1):
    nc = 2
    wid = lax.axis_index("s") * nc + lax.axis_index("c")
    row0 = wid * _ROWS_PER_W

    pltpu.sync_copy(pos_hbm, pos_v)
    pltpu.sync_copy(dt_hbm, dt_v)
    pltpu.sync_copy(s_hbm, s_v)

    def compute_p(c, _):
        v = pos_v[pl.ds(c * _L, _L)]
        for i in range(5):
            p_v[i, pl.ds(c * _L, _L)] = lax.div(v, jnp.int32(_R[i]))
        return 0

    lax.fori_loop(0, _T // _L, compute_p, 0)

    def clear(k, _):
        delta_v[k] = jnp.zeros((_L,), jnp.float32)
        return 0

    lax.fori_loop(0, _DPAD, clear, 0)

    o_iota = lax.iota(jnp.int32, _L)
    o_mul = o_iota * _T
    steps = [1024, 512, 256, 128, 64, 32, 16, 8, 4, 2, 1, 1]
    zero16 = jnp.zeros((_L,), jnp.float32)

    def one_row(row, slab_v):
        rowfull = jnp.full((_L,), 0, dtype=jnp.int32) + row
        for i in range(5):
            ifull = jnp.full((_L,), i, dtype=jnp.int32)
            prv = plsc.load_gather(p_v, [ifull, rowfull])
            for g in range(4):
                tgt = prv + s_v[pl.ds(g * _L, _L)]
                lo = jnp.zeros((_L,), jnp.int32)
                for st in steps:
                    cand = lo + st
                    pv = plsc.load_gather(p_v, [ifull, cand - 1])
                    lo = jnp.where(pv < tgt, cand, lo)
                for j in range(_L):
                    b = _lane_bcast(lo, j)
                    val = dt_v[i * 64 + g * _L + j]
                    plsc.addupdate_scatter(delta_v, [b, o_iota], val)

        def walk(col, carry):
            acc, colv = carry
            acc = acc + delta_v[col]
            delta_v[col] = zero16
            plsc.store_scatter(slab_v, [o_iota, colv], acc)
            return acc, colv + 1

        lax.fori_loop(0, _T, walk,
                      (zero16, jnp.zeros((_L,), jnp.int32)), unroll=8)

    def do_pair(k, _):
        for u, (slab_v, sem) in enumerate(((slab0_v, sem0), (slab1_v, sem1))):
            row = row0 + 2 * k + u

            @pl.when(k > 0)
            def _wait():
                pltpu.make_async_copy(slab_v, out_hbm.at[:, row], sem).wait()

            one_row(row, slab_v)
            pltpu.async_copy(slab_v, out_hbm.at[:, row], sem)
        return 0

    lax.fori_loop(0, _ROWS_PER_W // 2, do_pair, 0)
    pltpu.make_async_copy(slab0_v, out_hbm.at[:, 0], sem0).wait()
    pltpu.make_async_copy(slab1_v, out_hbm.at[:, 0], sem1).wait()


@jax.jit
def kernel(pos, tables):
    u = jnp.take(tables, jnp.asarray(_B_OF_J), axis=1)
    d = jnp.concatenate([u[:, :1], u[:, 1:] - u[:, :-1]], axis=1)
    dt = d.reshape(5 * 64, _ODIM).astype(jnp.float32)
    s_arr = jnp.asarray(_S_ARR)

    mesh = plsc.VectorSubcoreMesh(core_axis_name="c", subcore_axis_name="s")
    f = functools.partial(
        pl.kernel,
        mesh=mesh,
        compiler_params=pltpu.CompilerParams(
            needs_layout_passes=False, use_tc_tiling_on_sc=False),
        out_type=jax.ShapeDtypeStruct((_ODIM, _T, _T), jnp.float32),
        scratch_types=[
            pltpu.VMEM((_T,), jnp.int32),
            pltpu.VMEM((5, _T), jnp.int32),
            pltpu.VMEM((5 * 64, _ODIM), jnp.float32),
            pltpu.VMEM((64,), jnp.int32),
            pltpu.VMEM((_DPAD, _ODIM), jnp.float32),
            pltpu.VMEM((_ODIM, _T), jnp.float32),
            pltpu.VMEM((_ODIM, _T), jnp.float32),
            pltpu.SemaphoreType.DMA,
            pltpu.SemaphoreType.DMA,
        ],
    )(_body)
    return f(pos.astype(jnp.int32), dt, s_arr)


# speedup vs baseline: 42.1334x; 1.0673x over previous
import functools
import math

import jax
import jax.numpy as jnp
import numpy as np
from jax import lax
from jax.experimental import pallas as pl
from jax.experimental.pallas import tpu as pltpu
from jax.experimental.pallas import tpu_sc as plsc

_R = [150, 600, 2400, 10000, 40000]
_NB = 64
_MAXD = 256
_ODIM = 16
_T = 2048
_L = 16
_NW = 32
_ROWS_PER_W = _T // _NW
_DPAD = _T + _L

_t = [0] * 32
for _k in range(1, 16):
    _t[_k] = _k
for _k in range(16, 32):
    _t[_k] = math.ceil(2 ** (_k / 4.0))

_s = np.zeros(64, dtype=np.int64)
_s[0] = -(1 << 30)
for _j in range(1, 32):
    _s[_j] = 1 - _t[32 - _j]
_s[32] = 0
for _j in range(33, 64):
    _s[_j] = _t[_j - 32]
_B_OF_J = np.array([63 - j if j < 32 else j - 32 for j in range(64)])

_SV, _BV, _NGRP = [], [], []
for _r in _R:
    _nm = (500000 - 1) // _r
    _js = max(_j for _j in range(64) if _s[_j] <= -_nm)
    _je = max(_j for _j in range(64) if _s[_j] <= _nm)
    _ss = list(_s[_js:_je + 1])
    _bs = list(_B_OF_J[_js:_je + 1])
    _ss[0] = -(1 << 30)
    _kp = -(-len(_ss) // _L) * _L
    _ss += [1 << 30] * (_kp - len(_ss))
    _bs += [_bs[-1]] * (_kp - len(_bs))
    _SV.append(np.array(_ss, dtype=np.int64))
    _BV.append(np.array(_bs))
    _NGRP.append(_kp // _L)
_S_ARR = np.concatenate(_SV).astype(np.int32)
_OFF = np.cumsum([0] + [_L * g for g in _NGRP])
_NRUN = int(_OFF[-1])


def _lane_bcast(vec, j):
    idx = jnp.full((_L, 1), j, dtype=jnp.int32)
    dnums = lax.GatherDimensionNumbers(
        offset_dims=(), collapsed_slice_dims=(0,), start_index_map=(0,))
    return lax.gather(vec, idx, dnums, (1,),
                      mode=lax.GatherScatterMode.PROMISE_IN_BOUNDS)


def _body(pos_hbm, dt_hbm, u_hbm, s_hbm, out_hbm, pos_v, p_v, dt_v, u_v, s_v,
          delta_v, slab0_v, slab1_v, sem0, sem1):
    nc = 2
    wid = lax.axis_index("s") * nc + lax.axis_index("c")
    row0 = wid * _ROWS_PER_W

    pltpu.sync_copy(pos_hbm, pos_v)
    pltpu.sync_copy(dt_hbm, dt_v)
    pltpu.sync_copy(u_hbm, u_v)
    pltpu.sync_copy(s_hbm, s_v)

    def compute_p(c, _):
        v = pos_v[pl.ds(c * _L, _L)]
        for i in range(5):
            p_v[i, pl.ds(c * _L, _L)] = lax.div(v, jnp.int32(_R[i]))
        return 0

    lax.fori_loop(0, _T // _L, compute_p, 0)

    def clear(k, _):
        delta_v[k] = jnp.zeros((_L,), jnp.float32)
        return 0

    lax.fori_loop(0, _DPAD, clear, 0)

    o_iota = lax.iota(jnp.int32, _L)
    o_mul = o_iota * _T
    steps = [1024, 512, 256, 128, 64, 32, 16, 8, 4, 2, 1, 1]
    zero16 = jnp.zeros((_L,), jnp.float32)

    n_chain = 4
    csz = _T // n_chain
    sg_all = [(i, g) for i in range(5) for g in range(_NGRP[i])]
    batches = [sg_all[:8], sg_all[8:]]

    def one_row(row, slab_v):
        rowfull = jnp.full((_L,), 0, dtype=jnp.int32) + row
        ifulls = [jnp.full((_L,), i, dtype=jnp.int32) for i in range(5)]
        prvs = [plsc.load_gather(p_v, [ifulls[i], rowfull]) for i in range(5)]
        los = {}
        for batch in batches:
            tgts = {}
            for (i, g) in batch:
                off = int(_OFF[i]) + g * _L
                tgts[(i, g)] = prvs[i] + s_v[pl.ds(off, _L)]
                los[(i, g)] = jnp.zeros((_L,), jnp.int32)
            for st in steps:
                idxs = {k: los[k] + (st - 1) for k in batch}
                pvs = {k: plsc.load_gather(p_v, [ifulls[k[0]], idxs[k]])
                       for k in batch}
                for k in batch:
                    los[k] = jnp.where(pvs[k] < tgts[k], los[k] + st, los[k])
            for (i, g) in batch:
                lo = los[(i, g)]
                for j in range(_L):
                    b = _lane_bcast(lo, j)
                    val = dt_v[int(_OFF[i]) + g * _L + j]
                    plsc.addupdate_scatter(delta_v, [b, o_iota], val)
        accs = [zero16]
        for h in range(1, n_chain):
            c0 = h * csz
            acc = zero16
            for i in range(5):
                cntv = (los[(i, 0)] <= c0 - 1).astype(jnp.int32)
                for g in range(1, _NGRP[i]):
                    cntv = cntv + (los[(i, g)] <= c0 - 1).astype(jnp.int32)
                cnt = jnp.sum(cntv)
                acc = acc + u_v[int(_OFF[i]) - 1 + cnt]
            accs.append(acc)

        def walk(t, carry):
            colv = carry[-1]
            outs = []
            for h, acc in enumerate(carry[:-1]):
                col = h * csz + t
                acc = acc + delta_v[col]
                delta_v[col] = zero16
                plsc.store_scatter(slab_v, [o_iota, colv + (h * csz)], acc)
                outs.append(acc)
            return (*outs, colv + 1)

        lax.fori_loop(0, csz, walk,
                      (*accs, jnp.zeros((_L,), jnp.int32)), unroll=4)

    def do_pair(k, _):
        for u, (slab_v, sem) in enumerate(((slab0_v, sem0), (slab1_v, sem1))):
            row = row0 + 2 * k + u

            @pl.when(k > 0)
            def _wait():
                pltpu.make_async_copy(slab_v, out_hbm.at[:, row], sem).wait()

            one_row(row, slab_v)
            pltpu.async_copy(slab_v, out_hbm.at[:, row], sem)
        return 0

    lax.fori_loop(0, _ROWS_PER_W // 2, do_pair, 0)
    pltpu.make_async_copy(slab0_v, out_hbm.at[:, 0], sem0).wait()
    pltpu.make_async_copy(slab1_v, out_hbm.at[:, 0], sem1).wait()


@jax.jit
def kernel(pos, tables):
    us, ds = [], []
    for i in range(5):
        ui = jnp.take(tables[i], jnp.asarray(_BV[i]), axis=0)
        us.append(ui)
        ds.append(jnp.concatenate([ui[:1], ui[1:] - ui[:-1]], axis=0))
    u = jnp.concatenate(us, axis=0).astype(jnp.float32)
    dt = jnp.concatenate(ds, axis=0).astype(jnp.float32)
    s_arr = jnp.asarray(_S_ARR)

    mesh = plsc.VectorSubcoreMesh(core_axis_name="c", subcore_axis_name="s")
    f = functools.partial(
        pl.kernel,
        mesh=mesh,
        compiler_params=pltpu.CompilerParams(
            needs_layout_passes=False, use_tc_tiling_on_sc=False),
        out_type=jax.ShapeDtypeStruct((_ODIM, _T, _T), jnp.float32),
        scratch_types=[
            pltpu.VMEM((_T,), jnp.int32),
            pltpu.VMEM((5, _T), jnp.int32),
            pltpu.VMEM((_NRUN, _ODIM), jnp.float32),
            pltpu.VMEM((_NRUN, _ODIM), jnp.float32),
            pltpu.VMEM((_NRUN,), jnp.int32),
            pltpu.VMEM((_DPAD, _ODIM), jnp.float32),
            pltpu.VMEM((_ODIM, _T), jnp.float32),
            pltpu.VMEM((_ODIM, _T), jnp.float32),
            pltpu.SemaphoreType.DMA,
            pltpu.SemaphoreType.DMA,
        ],
    )(_body)
    return f(pos.astype(jnp.int32), dt, u, s_arr)


# speedup vs baseline: 46.3874x; 1.1010x over previous
import functools
import math

import jax
import jax.numpy as jnp
import numpy as np
from jax import lax
from jax.experimental import pallas as pl
from jax.experimental.pallas import tpu as pltpu
from jax.experimental.pallas import tpu_sc as plsc

_R = [150, 600, 2400, 10000, 40000]
_NB = 64
_MAXD = 256
_ODIM = 16
_T = 2048
_L = 16
_NW = 32
_ROWS_PER_W = _T // _NW
_DPAD = _T + _L

_t = [0] * 32
for _k in range(1, 16):
    _t[_k] = _k
for _k in range(16, 32):
    _t[_k] = math.ceil(2 ** (_k / 4.0))

_s = np.zeros(64, dtype=np.int64)
_s[0] = -(1 << 30)
for _j in range(1, 32):
    _s[_j] = 1 - _t[32 - _j]
_s[32] = 0
for _j in range(33, 64):
    _s[_j] = _t[_j - 32]
_B_OF_J = np.array([63 - j if j < 32 else j - 32 for j in range(64)])

_SV, _BV, _NGRP = [], [], []
for _r in _R:
    _nm = (500000 - 1) // _r
    _js = max(_j for _j in range(64) if _s[_j] <= -_nm)
    _je = max(_j for _j in range(64) if _s[_j] <= _nm)
    _ss = list(_s[_js:_je + 1])
    _bs = list(_B_OF_J[_js:_je + 1])
    _ss[0] = -(1 << 30)
    _kp = -(-len(_ss) // _L) * _L
    _ss += [1 << 30] * (_kp - len(_ss))
    _bs += [_bs[-1]] * (_kp - len(_bs))
    _SV.append(np.array(_ss, dtype=np.int64))
    _BV.append(np.array(_bs))
    _NGRP.append(_kp // _L)
_S_ARR = np.concatenate(_SV).astype(np.int32)
_OFF = np.cumsum([0] + [_L * g for g in _NGRP])
_NRUN = int(_OFF[-1])


def _lane_bcast(vec, j):
    idx = jnp.full((_L, 1), j, dtype=jnp.int32)
    dnums = lax.GatherDimensionNumbers(
        offset_dims=(), collapsed_slice_dims=(0,), start_index_map=(0,))
    return lax.gather(vec, idx, dnums, (1,),
                      mode=lax.GatherScatterMode.PROMISE_IN_BOUNDS)


def _body(pos_hbm, dt_hbm, u_hbm, s_hbm, out_hbm, pos_v, p_v, dt_v, u_v, s_v,
          delta_v, slab0_v, slab1_v, sem0, sem1):
    nc = 2
    wid = lax.axis_index("s") * nc + lax.axis_index("c")
    row0 = wid * _ROWS_PER_W

    pltpu.sync_copy(pos_hbm, pos_v)
    pltpu.sync_copy(dt_hbm, dt_v)
    pltpu.sync_copy(u_hbm, u_v)
    pltpu.sync_copy(s_hbm, s_v)

    def compute_p(c, _):
        v = pos_v[pl.ds(c * _L, _L)]
        for i in range(5):
            p_v[i, pl.ds(c * _L, _L)] = lax.div(v, jnp.int32(_R[i]))
        return 0

    lax.fori_loop(0, _T // _L, compute_p, 0)

    def clear(k, _):
        delta_v[k] = jnp.zeros((_L,), jnp.float32)
        return 0

    lax.fori_loop(0, _DPAD, clear, 0)

    o_iota = lax.iota(jnp.int32, _L)
    o_mul = o_iota * _T
    steps = [1024, 512, 256, 128, 64, 32, 16, 8, 4, 2, 1, 1]
    zero16 = jnp.zeros((_L,), jnp.float32)

    n_chain = 4
    csz = _T // n_chain
    sg_all = [(i, g) for i in range(5) for g in range(_NGRP[i])]
    batches = [sg_all[:8], sg_all[8:]]

    def one_row(row, slab_v):
        rowfull = jnp.full((_L,), 0, dtype=jnp.int32) + row
        ifulls = [jnp.full((_L,), i, dtype=jnp.int32) for i in range(5)]
        prvs = [plsc.load_gather(p_v, [ifulls[i], rowfull]) for i in range(5)]
        los = {}
        for batch in batches:
            tgts = {}
            for (i, g) in batch:
                off = int(_OFF[i]) + g * _L
                tgts[(i, g)] = prvs[i] + s_v[pl.ds(off, _L)]
                los[(i, g)] = jnp.zeros((_L,), jnp.int32)
            for st in steps:
                idxs = {k: los[k] + (st - 1) for k in batch}
                pvs = {k: plsc.load_gather(p_v, [ifulls[k[0]], idxs[k]])
                       for k in batch}
                for k in batch:
                    los[k] = jnp.where(pvs[k] < tgts[k], los[k] + st, los[k])
            for (i, g) in batch:
                lo = los[(i, g)]
                for j in range(_L):
                    b = _lane_bcast(lo, j)
                    val = dt_v[int(_OFF[i]) + g * _L + j]
                    plsc.addupdate_scatter(delta_v, [b, o_iota], val)
        accs = [zero16]
        for h in range(1, n_chain):
            c0 = h * csz
            acc = zero16
            for i in range(5):
                cntv = (los[(i, 0)] <= c0 - 1).astype(jnp.int32)
                for g in range(1, _NGRP[i]):
                    cntv = cntv + (los[(i, g)] <= c0 - 1).astype(jnp.int32)
                cnt = jnp.sum(cntv)
                acc = acc + u_v[int(_OFF[i]) - 1 + cnt]
            accs.append(acc)

        zidx = jnp.zeros((_L,), jnp.int32)

        def walk(t, carry):
            addrv = carry[-1]
            ds = [delta_v[h * csz + t] for h in range(n_chain)]
            naccs = [carry[h] + ds[h] for h in range(n_chain)]
            for h in range(n_chain):
                delta_v[h * csz + t] = zero16
            addrs = [addrv + (h * csz) if h else addrv for h in range(n_chain)]
            for h in range(n_chain):
                plsc.store_scatter(slab_v, [zidx, addrs[h]], naccs[h])
            return (*naccs, addrv + 1)

        lax.fori_loop(0, csz, walk, (*accs, o_mul), unroll=4)

    def do_pair(k, _):
        for u, (slab_v, sem) in enumerate(((slab0_v, sem0), (slab1_v, sem1))):
            row = row0 + 2 * k + u

            @pl.when(k > 0)
            def _wait():
                pltpu.make_async_copy(slab_v, out_hbm.at[:, row], sem).wait()

            one_row(row, slab_v)
            pltpu.async_copy(slab_v, out_hbm.at[:, row], sem)
        return 0

    lax.fori_loop(0, _ROWS_PER_W // 2, do_pair, 0)
    pltpu.make_async_copy(slab0_v, out_hbm.at[:, 0], sem0).wait()
    pltpu.make_async_copy(slab1_v, out_hbm.at[:, 0], sem1).wait()


@jax.jit
def kernel(pos, tables):
    us, ds = [], []
    for i in range(5):
        ui = jnp.take(tables[i], jnp.asarray(_BV[i]), axis=0)
        us.append(ui)
        ds.append(jnp.concatenate([ui[:1], ui[1:] - ui[:-1]], axis=0))
    u = jnp.concatenate(us, axis=0).astype(jnp.float32)
    dt = jnp.concatenate(ds, axis=0).astype(jnp.float32)
    s_arr = jnp.asarray(_S_ARR)

    mesh = plsc.VectorSubcoreMesh(core_axis_name="c", subcore_axis_name="s")
    f = functools.partial(
        pl.kernel,
        mesh=mesh,
        compiler_params=pltpu.CompilerParams(
            needs_layout_passes=False, use_tc_tiling_on_sc=False),
        out_type=jax.ShapeDtypeStruct((_ODIM, _T, _T), jnp.float32),
        scratch_types=[
            pltpu.VMEM((_T,), jnp.int32),
            pltpu.VMEM((5, _T), jnp.int32),
            pltpu.VMEM((_NRUN, _ODIM), jnp.float32),
            pltpu.VMEM((_NRUN, _ODIM), jnp.float32),
            pltpu.VMEM((_NRUN,), jnp.int32),
            pltpu.VMEM((_DPAD, _ODIM), jnp.float32),
            pltpu.VMEM((_ODIM, _T), jnp.float32),
            pltpu.VMEM((_ODIM, _T), jnp.float32),
            pltpu.SemaphoreType.DMA,
            pltpu.SemaphoreType.DMA,
        ],
    )(_body)
    return f(pos.astype(jnp.int32), dt, u, s_arr)


# speedup vs baseline: 102.0411x; 2.1998x over previous
import functools
import math

import jax
import jax.numpy as jnp
import numpy as np
from jax import lax
from jax.experimental import pallas as pl
from jax.experimental.pallas import tpu as pltpu
from jax.experimental.pallas import tpu_sc as plsc

_R = [150, 600, 2400, 10000, 40000]
_NB = 64
_MAXD = 256
_ODIM = 16
_T = 2048
_L = 16
_NW = 32
_ROWS_PER_W = _T // _NW
_DPAD = _T + _L
_CSKEW = _ODIM + 1

_t = [0] * 32
for _k in range(1, 16):
    _t[_k] = _k
for _k in range(16, 32):
    _t[_k] = math.ceil(2 ** (_k / 4.0))

_s = np.zeros(64, dtype=np.int64)
_s[0] = -(1 << 30)
for _j in range(1, 32):
    _s[_j] = 1 - _t[32 - _j]
_s[32] = 0
for _j in range(33, 64):
    _s[_j] = _t[_j - 32]
_B_OF_J = np.array([63 - j if j < 32 else j - 32 for j in range(64)])

_SV, _BV, _NGRP = [], [], []
for _r in _R:
    _nm = (500000 - 1) // _r
    _js = max(_j for _j in range(64) if _s[_j] <= -_nm)
    _je = max(_j for _j in range(64) if _s[_j] <= _nm)
    _ss = list(_s[_js:_je + 1])
    _bs = list(_B_OF_J[_js:_je + 1])
    _ss[0] = -(1 << 30)
    _kp = -(-len(_ss) // _L) * _L
    _ss += [1 << 30] * (_kp - len(_ss))
    _bs += [_bs[-1]] * (_kp - len(_bs))
    _SV.append(np.array(_ss, dtype=np.int64))
    _BV.append(np.array(_bs))
    _NGRP.append(_kp // _L)
_S_ARR = np.concatenate(_SV).astype(np.int32)
_OFF = np.cumsum([0] + [_L * g for g in _NGRP])
_NRUN = int(_OFF[-1])


def _lane_bcast(vec, j):
    idx = jnp.full((_L, 1), j, dtype=jnp.int32)
    dnums = lax.GatherDimensionNumbers(
        offset_dims=(), collapsed_slice_dims=(0,), start_index_map=(0,))
    return lax.gather(vec, idx, dnums, (1,),
                      mode=lax.GatherScatterMode.PROMISE_IN_BOUNDS)


def _body(pos_hbm, dt_hbm, u_hbm, s_hbm, out_hbm, pos_v, p_v, dt_v, u_v, s_v,
          delta_v, slab_v, sem0):
    nc = 2
    wid = lax.axis_index("s") * nc + lax.axis_index("c")
    row0 = wid * _ROWS_PER_W

    pltpu.sync_copy(pos_hbm, pos_v)
    pltpu.sync_copy(dt_hbm, dt_v)
    pltpu.sync_copy(u_hbm, u_v)
    pltpu.sync_copy(s_hbm, s_v)

    def compute_p(c, _):
        v = pos_v[pl.ds(c * _L, _L)]
        for i in range(5):
            p_v[i, pl.ds(c * _L, _L)] = lax.div(v, jnp.int32(_R[i]))
        return 0

    lax.fori_loop(0, _T // _L, compute_p, 0)

    def clear(k, _):
        delta_v[k, pl.ds(0, _L)] = jnp.zeros((_L,), jnp.float32)
        return 0

    lax.fori_loop(0, _DPAD, clear, 0)

    o_iota = lax.iota(jnp.int32, _L)
    steps = [1024, 512, 256, 128, 64, 32, 16, 8, 4, 2, 1, 1]
    zero16 = jnp.zeros((_L,), jnp.float32)

    n_chain = 4
    csz = _T // n_chain
    sg_all = [(i, g) for i in range(5) for g in range(_NGRP[i])]
    batches = [sg_all[:8], sg_all[8:]]

    def one_row(row, ridx):
        rowfull = jnp.full((_L,), 0, dtype=jnp.int32) + row
        ifulls = [jnp.full((_L,), i, dtype=jnp.int32) for i in range(5)]
        prvs = [plsc.load_gather(p_v, [ifulls[i], rowfull]) for i in range(5)]
        los = {}
        for batch in batches:
            tgts = {}
            for (i, g) in batch:
                off = int(_OFF[i]) + g * _L
                tgts[(i, g)] = prvs[i] + s_v[pl.ds(off, _L)]
                los[(i, g)] = jnp.zeros((_L,), jnp.int32)
            for st in steps:
                idxs = {k: los[k] + (st - 1) for k in batch}
                pvs = {k: plsc.load_gather(p_v, [ifulls[k[0]], idxs[k]])
                       for k in batch}
                for k in batch:
                    los[k] = jnp.where(pvs[k] < tgts[k], los[k] + st, los[k])
            for (i, g) in batch:
                lo = los[(i, g)]
                for j in range(_L):
                    b = _lane_bcast(lo, j)
                    val = dt_v[int(_OFF[i]) + g * _L + j]
                    plsc.addupdate_scatter(delta_v, [b, o_iota], val)
        accs = [zero16]
        for h in range(1, n_chain):
            c0 = h * csz
            acc = zero16
            for i in range(5):
                cntv = (los[(i, 0)] <= c0 - 1).astype(jnp.int32)
                for g in range(1, _NGRP[i]):
                    cntv = cntv + (los[(i, g)] <= c0 - 1).astype(jnp.int32)
                cnt = jnp.sum(cntv)
                acc = acc + u_v[int(_OFF[i]) - 1 + cnt]
            accs.append(acc)

        def walk(t, carry):
            ds = [delta_v[h * csz + t, pl.ds(0, _L)] for h in range(n_chain)]
            naccs = [carry[h] + ds[h] for h in range(n_chain)]
            for h in range(n_chain):
                delta_v[h * csz + t, pl.ds(0, _L)] = naccs[h]
            return tuple(naccs)

        lax.fori_loop(0, csz, walk, tuple(accs), unroll=4)

        @pl.when(ridx > 0)
        def _wait():
            pltpu.make_async_copy(slab_v, out_hbm.at[:, row], sem0).wait()

        ofulls = [jnp.full((_L,), o, dtype=jnp.int32) for o in range(_ODIM)]

        gs = [plsc.load_gather(delta_v, [o_iota, ofulls[o]])
              for o in range(_ODIM)]
        for o in range(_ODIM):
            slab_v[o, pl.ds(0, _L)] = gs[o]

        def transpose(b, _):
            c0 = b * _L
            cvec = o_iota + c0
            gs = [plsc.load_gather(delta_v, [cvec, ofulls[o]])
                  for o in range(_ODIM)]
            for o in range(_ODIM):
                slab_v[o, pl.ds(c0, _L)] = gs[o]
            for l in range(_L):
                delta_v[c0 - _L + l, pl.ds(0, _L)] = zero16
            return 0

        lax.fori_loop(1, _T // _L, transpose, 0)
        for l in range(_L):
            delta_v[_T - _L + l, pl.ds(0, _L)] = zero16

    def do_row(k, _):
        row = row0 + k
        one_row(row, k)
        pltpu.async_copy(slab_v, out_hbm.at[:, row], sem0)
        return 0

    lax.fori_loop(0, _ROWS_PER_W, do_row, 0)
    pltpu.make_async_copy(slab_v, out_hbm.at[:, 0], sem0).wait()


@jax.jit
def kernel(pos, tables):
    us, ds = [], []
    for i in range(5):
        ui = jnp.take(tables[i], jnp.asarray(_BV[i]), axis=0)
        us.append(ui)
        ds.append(jnp.concatenate([ui[:1], ui[1:] - ui[:-1]], axis=0))
    u = jnp.concatenate(us, axis=0).astype(jnp.float32)
    dt = jnp.concatenate(ds, axis=0).astype(jnp.float32)
    s_arr = jnp.asarray(_S_ARR)

    mesh = plsc.VectorSubcoreMesh(core_axis_name="c", subcore_axis_name="s")
    f = functools.partial(
        pl.kernel,
        mesh=mesh,
        compiler_params=pltpu.CompilerParams(
            needs_layout_passes=False, use_tc_tiling_on_sc=False),
        out_type=jax.ShapeDtypeStruct((_ODIM, _T, _T), jnp.float32),
        scratch_types=[
            pltpu.VMEM((_T,), jnp.int32),
            pltpu.VMEM((5, _T), jnp.int32),
            pltpu.VMEM((_NRUN, _ODIM), jnp.float32),
            pltpu.VMEM((_NRUN, _ODIM), jnp.float32),
            pltpu.VMEM((_NRUN,), jnp.int32),
            pltpu.VMEM((_DPAD, _CSKEW), jnp.float32),
            pltpu.VMEM((_ODIM, _T), jnp.float32),
            pltpu.SemaphoreType.DMA,
        ],
    )(_body)
    return f(pos.astype(jnp.int32), dt, u, s_arr)


# speedup vs baseline: 113.2684x; 1.1100x over previous
import functools
import math

import jax
import jax.numpy as jnp
import numpy as np
from jax import lax
from jax.experimental import pallas as pl
from jax.experimental.pallas import tpu as pltpu
from jax.experimental.pallas import tpu_sc as plsc

_R = [150, 600, 2400, 10000, 40000]
_NB = 64
_MAXD = 256
_ODIM = 16
_T = 2048
_L = 16
_NW = 32
_ROWS_PER_W = _T // _NW
_DPAD = _T + _L
_CSKEW = _ODIM + 1

_t = [0] * 32
for _k in range(1, 16):
    _t[_k] = _k
for _k in range(16, 32):
    _t[_k] = math.ceil(2 ** (_k / 4.0))

_s = np.zeros(64, dtype=np.int64)
_s[0] = -(1 << 30)
for _j in range(1, 32):
    _s[_j] = 1 - _t[32 - _j]
_s[32] = 0
for _j in range(33, 64):
    _s[_j] = _t[_j - 32]
_B_OF_J = np.array([63 - j if j < 32 else j - 32 for j in range(64)])

_SV, _BV, _NGRP = [], [], []
for _r in _R:
    _nm = (500000 - 1) // _r
    _js = max(_j for _j in range(64) if _s[_j] <= -_nm)
    _je = max(_j for _j in range(64) if _s[_j] <= _nm)
    _ss = list(_s[_js:_je + 1])
    _bs = list(_B_OF_J[_js:_je + 1])
    _ss[0] = -(1 << 30)
    _kp = -(-len(_ss) // _L) * _L
    _ss += [1 << 30] * (_kp - len(_ss))
    _bs += [_bs[-1]] * (_kp - len(_bs))
    _SV.append(np.array(_ss, dtype=np.int64))
    _BV.append(np.array(_bs))
    _NGRP.append(_kp // _L)
_S_ARR = np.concatenate(_SV).astype(np.int32)
_OFF = np.cumsum([0] + [_L * g for g in _NGRP])
_NRUN = int(_OFF[-1])


def _lane_bcast(vec, j):
    idx = jnp.full((_L, 1), j, dtype=jnp.int32)
    dnums = lax.GatherDimensionNumbers(
        offset_dims=(), collapsed_slice_dims=(0,), start_index_map=(0,))
    return lax.gather(vec, idx, dnums, (1,),
                      mode=lax.GatherScatterMode.PROMISE_IN_BOUNDS)


def _body(pos_hbm, dt_hbm, u_hbm, s_hbm, out_hbm, pos_v, p_v, dt_v, u_v, s_v,
          delta_v, slab_v, sem0):
    nc = 2
    wid = lax.axis_index("s") * nc + lax.axis_index("c")
    row0 = wid * _ROWS_PER_W

    pltpu.sync_copy(pos_hbm, pos_v)
    pltpu.sync_copy(dt_hbm, dt_v)
    pltpu.sync_copy(u_hbm, u_v)
    pltpu.sync_copy(s_hbm, s_v)

    def compute_p(c, _):
        v = pos_v[pl.ds(c * _L, _L)]
        for i in range(5):
            p_v[i, pl.ds(c * _L, _L)] = lax.div(v, jnp.int32(_R[i]))
        return 0

    lax.fori_loop(0, _T // _L, compute_p, 0)

    def clear(k, _):
        delta_v[k, pl.ds(0, _L)] = jnp.zeros((_L,), jnp.float32)
        return 0

    lax.fori_loop(0, _DPAD, clear, 0)

    o_iota = lax.iota(jnp.int32, _L)
    steps = [1024, 512, 256, 128, 64, 32, 16, 8, 4, 2, 1, 1]
    zero16 = jnp.zeros((_L,), jnp.float32)

    n_chain = 4
    csz = _T // n_chain
    sg_all = [(i, g) for i in range(5) for g in range(_NGRP[i])]
    batches = [sg_all[:8], sg_all[8:]]

    def one_row(row, ridx):
        rowfull = jnp.full((_L,), 0, dtype=jnp.int32) + row
        ifulls = [jnp.full((_L,), i, dtype=jnp.int32) for i in range(5)]
        prvs = [plsc.load_gather(p_v, [ifulls[i], rowfull]) for i in range(5)]
        los = {}
        for batch in batches:
            tgts = {}
            for (i, g) in batch:
                off = int(_OFF[i]) + g * _L
                tgts[(i, g)] = prvs[i] + s_v[pl.ds(off, _L)]
                los[(i, g)] = jnp.zeros((_L,), jnp.int32)
            for st in steps:
                idxs = {k: los[k] + (st - 1) for k in batch}
                pvs = {k: plsc.load_gather(p_v, [ifulls[k[0]], idxs[k]])
                       for k in batch}
                for k in batch:
                    los[k] = jnp.where(pvs[k] < tgts[k], los[k] + st, los[k])
            for (i, g) in batch:
                lo = los[(i, g)]
                bs = [_lane_bcast(lo, j) for j in range(_L)]
                vals = [dt_v[int(_OFF[i]) + g * _L + j] for j in range(_L)]
                for j in range(_L):
                    plsc.addupdate_scatter(delta_v, [bs[j], o_iota], vals[j])
        accs = [zero16]
        for h in range(1, n_chain):
            c0 = h * csz
            acc = zero16
            for i in range(5):
                cntv = (los[(i, 0)] <= c0 - 1).astype(jnp.int32)
                for g in range(1, _NGRP[i]):
                    cntv = cntv + (los[(i, g)] <= c0 - 1).astype(jnp.int32)
                cnt = jnp.sum(cntv)
                acc = acc + u_v[int(_OFF[i]) - 1 + cnt]
            accs.append(acc)

        def walk(t, carry):
            ds = [delta_v[h * csz + t, pl.ds(0, _L)] for h in range(n_chain)]
            naccs = [carry[h] + ds[h] for h in range(n_chain)]
            for h in range(n_chain):
                delta_v[h * csz + t, pl.ds(0, _L)] = naccs[h]
            return tuple(naccs)

        lax.fori_loop(0, csz, walk, tuple(accs), unroll=4)

        @pl.when(ridx > 0)
        def _wait():
            pltpu.make_async_copy(slab_v, out_hbm.at[:, row], sem0).wait()

        ofulls = [jnp.full((_L,), o, dtype=jnp.int32) for o in range(_ODIM)]

        gs = [plsc.load_gather(delta_v, [o_iota, ofulls[o]])
              for o in range(_ODIM)]
        for o in range(_ODIM):
            slab_v[o, pl.ds(0, _L)] = gs[o]

        def transpose(b, _):
            c0 = b * _L
            cvec = o_iota + c0
            gs = [plsc.load_gather(delta_v, [cvec, ofulls[o]])
                  for o in range(_ODIM)]
            for o in range(_ODIM):
                slab_v[o, pl.ds(c0, _L)] = gs[o]
            for l in range(_L):
                delta_v[c0 - _L + l, pl.ds(0, _L)] = zero16
            return 0

        lax.fori_loop(1, _T // _L, transpose, 0)
        for l in range(_L):
            delta_v[_T - _L + l, pl.ds(0, _L)] = zero16

    def do_row(k, _):
        row = row0 + k
        one_row(row, k)
        pltpu.async_copy(slab_v, out_hbm.at[:, row], sem0)
        return 0

    lax.fori_loop(0, _ROWS_PER_W, do_row, 0)
    pltpu.make_async_copy(slab_v, out_hbm.at[:, 0], sem0).wait()


@jax.jit
def kernel(pos, tables):
    us, ds = [], []
    for i in range(5):
        ui = jnp.take(tables[i], jnp.asarray(_BV[i]), axis=0)
        us.append(ui)
        ds.append(jnp.concatenate([ui[:1], ui[1:] - ui[:-1]], axis=0))
    u = jnp.concatenate(us, axis=0).astype(jnp.float32)
    dt = jnp.concatenate(ds, axis=0).astype(jnp.float32)
    s_arr = jnp.asarray(_S_ARR)

    mesh = plsc.VectorSubcoreMesh(core_axis_name="c", subcore_axis_name="s")
    f = functools.partial(
        pl.kernel,
        mesh=mesh,
        compiler_params=pltpu.CompilerParams(
            needs_layout_passes=False, use_tc_tiling_on_sc=False),
        out_type=jax.ShapeDtypeStruct((_ODIM, _T, _T), jnp.float32),
        scratch_types=[
            pltpu.VMEM((_T,), jnp.int32),
            pltpu.VMEM((5, _T), jnp.int32),
            pltpu.VMEM((_NRUN, _ODIM), jnp.float32),
            pltpu.VMEM((_NRUN, _ODIM), jnp.float32),
            pltpu.VMEM((_NRUN,), jnp.int32),
            pltpu.VMEM((_DPAD, _CSKEW), jnp.float32),
            pltpu.VMEM((_ODIM, _T), jnp.float32),
            pltpu.SemaphoreType.DMA,
        ],
    )(_body)
    return f(pos.astype(jnp.int32), dt, u, s_arr)
